# baseline (speedup 1.0000x reference)
import functools

import jax
import jax.numpy as jnp
from jax import lax
from jax.experimental import pallas as pl
from jax.experimental.pallas import tpu as pltpu
from jax.experimental.pallas import tpu_sc as plsc

_HIDDEN = 1024
_NUM_EXPERTS = 16384
_KEY_DIMS = 128
_K = 8
_NUM_HEADS = 8
_NUM_KEYS = 128
_KH = _NUM_HEADS * _K
_KC = 32
_NCH = _KH // _KC
_L = 16
_DC = _HIDDEN // _L


def _gelu16(x):
    z = jnp.abs(x) * 0.7071067811865476
    t = 1.0 / (1.0 + 0.3275911 * z)
    poly = ((((1.061405429 * t - 1.453152027) * t + 1.421413741) * t
             - 0.284496736) * t + 0.254829592) * t
    erf_abs = 1.0 - poly * jnp.exp(-z * z)
    erf = jnp.where(x >= 0.0, erf_abs, -erf_abs)
    return x * 0.5 * (1.0 + erf)


_NUM_SC = 2
_NUM_SUBCORES = 16
_DC32 = _HIDDEN // 32


def _bf16_split(w):
    w = plsc.bitcast(w, jnp.uint32)
    a = plsc.bitcast(w << 16, jnp.float32)
    b = plsc.bitcast(w & jnp.uint32(0xFFFF0000), jnp.float32)
    return a, b


def _make_sc_combine(n_tokens):
    nw = _NUM_SC * _NUM_SUBCORES
    tpw = n_tokens // nw
    mesh = plsc.VectorSubcoreMesh(core_axis_name="c", subcore_axis_name="s",
                                  num_cores=_NUM_SC, num_subcores=_NUM_SUBCORES)

    @functools.partial(
        pl.kernel,
        out_type=jax.ShapeDtypeStruct((n_tokens, _HIDDEN), jnp.float32),
        mesh=mesh,
        compiler_params=pltpu.CompilerParams(needs_layout_passes=False),
        scratch_types=[
            pltpu.VMEM((2, _HIDDEN), jnp.float32),
            pltpu.VMEM((2, _KH), jnp.int32),
            pltpu.VMEM((2, _KH), jnp.float32),
            pltpu.VMEM((_KH,), jnp.float32),
            pltpu.VMEM((_KH,), jnp.float32),
            pltpu.VMEM((_KC, _HIDDEN // 2), jnp.int32),
            pltpu.VMEM((_KC, _HIDDEN // 2), jnp.int32),
            pltpu.VMEM((2, _HIDDEN), jnp.float32),
            pltpu.SemaphoreType.DMA,
            pltpu.SemaphoreType.DMA,
            pltpu.SemaphoreType.DMA,
            pltpu.SemaphoreType.DMA,
        ],
    )
    def body(x_hbm, idx_hbm, s_hbm, down_hbm, up_hbm, out_hbm,
             x2, i2, s2, t_v, act_v, rowA, rowB, acc2,
             semA, semB, semP, semO):
        wid = lax.axis_index("s") * _NUM_SC + lax.axis_index("c")
        base = wid * tpw
        lane0 = lax.iota(jnp.int32, _L) == 0

        def wait_row(sem, row):
            pltpu.make_async_copy(down_hbm.at[pl.ds(0, _KC)], row, sem).wait()

        def dots(row, xbuf, par, h):
            for g in range(_KC // 8):

                def dot_body(dj, accs, g=g):
                    xa = xbuf[par, pl.ds(dj * 32, _L)]
                    xb = xbuf[par, pl.ds(dj * 32 + _L, _L)]
                    out = []
                    for r in range(8):
                        ra, rb = _bf16_split(row[g * 8 + r, pl.ds(dj * _L, _L)])
                        out.append(accs[r] + xa * ra + xb * rb)
                    return tuple(out)

                accs = lax.fori_loop(
                    0, _DC32, dot_body,
                    tuple(jnp.zeros((_L,), jnp.float32) for _ in range(8)))
                for r in range(8):
                    k = h * _KC + g * 8 + r
                    t = jnp.sum(accs[r])
                    plsc.store_scatter(
                        t_v, [jnp.full((_L,), k, jnp.int32)],
                        jnp.full((_L,), t, jnp.float32), mask=lane0)

        def combine(row, par, h):
            act_a = act_v[pl.ds(h * _KC, _L)]
            act_b = act_v[pl.ds(h * _KC + _L, _L)]

            def comb_body(dc, _, h=h, act_a=act_a, act_b=act_b):
                sla = pl.ds(dc * 32, _L)
                slb = pl.ds(dc * 32 + _L, _L)
                if h == 0:
                    aa = jnp.zeros((_L,), jnp.float32)
                    ab = jnp.zeros((_L,), jnp.float32)
                else:
                    aa = acc2[par, sla]
                    ab = acc2[par, slb]
                for kk in range(_L):
                    ra, rb = _bf16_split(row[kk, pl.ds(dc * _L, _L)])
                    aa = aa + act_a[kk] * ra
                    ab = ab + act_a[kk] * rb
                for kk in range(_L):
                    ra, rb = _bf16_split(row[_L + kk, pl.ds(dc * _L, _L)])
                    aa = aa + act_b[kk] * ra
                    ab = ab + act_b[kk] * rb
                acc2[par, sla] = aa
                acc2[par, slb] = ab
                return 0

            lax.fori_loop(0, _DC32, comb_body, 0)

        pltpu.sync_copy(x_hbm.at[base], x2.at[0])
        pltpu.sync_copy(s_hbm.at[base], s2.at[0])
        pltpu.sync_copy(idx_hbm.at[base], i2.at[0])
        pltpu.async_copy(down_hbm.at[i2.at[0, pl.ds(0, _KC)]], rowA, semA)
        pltpu.async_copy(down_hbm.at[i2.at[0, pl.ds(_KC, _KC)]], rowB, semB)

        def token_body(i, _):
            par = lax.rem(i, 2)
            nxt = 1 - par
            tok = base + i
            tok1 = base + jnp.minimum(i + 1, tpw - 1)

            pltpu.async_copy(x_hbm.at[tok1], x2.at[nxt], semP)
            pltpu.async_copy(s_hbm.at[tok1], s2.at[nxt], semP)
            pltpu.async_copy(idx_hbm.at[tok1], i2.at[nxt], semP)

            wait_row(semA, rowA)
            dots(rowA, x2, par, 0)
            pltpu.async_copy(up_hbm.at[i2.at[par, pl.ds(0, _KC)]], rowA, semA)
            wait_row(semB, rowB)
            dots(rowB, x2, par, 1)
            pltpu.async_copy(up_hbm.at[i2.at[par, pl.ds(_KC, _KC)]], rowB,
                             semB)

            for c in range(_KH // _L):
                sl = pl.ds(c * _L, _L)
                s16 = s2[par, sl]
                sig = 1.0 / (1.0 + jnp.exp(-s16))
                act_v[sl] = sig * _gelu16(t_v[sl])

            pltpu.make_async_copy(x_hbm.at[tok1], x2.at[nxt], semP).wait()
            pltpu.make_async_copy(s_hbm.at[tok1], s2.at[nxt], semP).wait()
            pltpu.make_async_copy(idx_hbm.at[tok1], i2.at[nxt], semP).wait()

            @pl.when(i >= 2)
            def _():
                pltpu.make_async_copy(acc2.at[par], out_hbm.at[tok],
                                      semO).wait()

            wait_row(semA, rowA)
            combine(rowA, par, 0)
            pltpu.async_copy(down_hbm.at[i2.at[nxt, pl.ds(0, _KC)]], rowA,
                             semA)
            wait_row(semB, rowB)
            combine(rowB, par, 1)
            pltpu.async_copy(down_hbm.at[i2.at[nxt, pl.ds(_KC, _KC)]], rowB,
                             semB)

            pltpu.async_copy(acc2.at[par], out_hbm.at[tok], semO)
            return 0

        lax.fori_loop(0, tpw, token_body, 0)

        wait_row(semA, rowA)
        wait_row(semB, rowB)
        pltpu.make_async_copy(acc2.at[0], out_hbm.at[base], semO).wait()
        pltpu.make_async_copy(acc2.at[1], out_hbm.at[base], semO).wait()

    return body


_TBLK = 256


def _top8_iter(s, iota_l, nl):
    vals, lanes = [], []
    for _ in range(_K):
        m = jnp.max(s, axis=1, keepdims=True)
        am = jnp.min(jnp.where(s == m, iota_l, nl), axis=1, keepdims=True)
        vals.append(m)
        lanes.append(am)
        s = jnp.where(iota_l == am, -1e30, s)
    return jnp.concatenate(vals, axis=1), jnp.concatenate(lanes, axis=1)


def _routing_body(x_ref, g_ref, b_ref, wq_ref, kt_ref, s_out, i_out):
    rsqrt_1p = 0.9999950000374997
    x = x_ref[...] * (g_ref[...] * rsqrt_1p) + b_ref[...]
    h_mat = jnp.dot(x, wq_ref[...], preferred_element_type=jnp.float32,
                    precision=lax.Precision.DEFAULT)

    iota128 = lax.broadcasted_iota(jnp.int32, (_TBLK, _NUM_KEYS), 1)
    iota64 = lax.broadcasted_iota(jnp.int32, (_TBLK, _K * _K), 1)
    col = lax.broadcasted_iota(jnp.int32, (_K, _K * _K), 1)
    row = lax.broadcasted_iota(jnp.int32, (_K, _K * _K), 0)
    Rm = (col // _K == row).astype(jnp.float32)
    Tm = (col % _K == row).astype(jnp.float32)

    s_cols, i_cols = [], []
    for hd in range(_NUM_HEADS):
        per_plane = []
        for p in range(2):
            q_ph = h_mat[:, p * _NUM_HEADS * _KEY_DIMS + hd * _KEY_DIMS:
                         p * _NUM_HEADS * _KEY_DIMS + (hd + 1) * _KEY_DIMS]
            kslab = kt_ref[pl.ds((p * _NUM_HEADS + hd) * _KEY_DIMS,
                                 _KEY_DIMS), :]
            sim = jnp.dot(q_ph, kslab, preferred_element_type=jnp.float32,
                          precision=lax.Precision.DEFAULT)
            per_plane.append(_top8_iter(sim, iota128, _NUM_KEYS))
        (sx, ix), (sy, iy) = per_plane
        all_s = (jnp.dot(sx, Rm, preferred_element_type=jnp.float32, precision=lax.Precision.HIGHEST)
                 + jnp.dot(sy, Tm, preferred_element_type=jnp.float32, precision=lax.Precision.HIGHEST))
        eidf = (jnp.dot(ix.astype(jnp.float32), Rm,
                        preferred_element_type=jnp.float32, precision=lax.Precision.HIGHEST) * _NUM_KEYS
                + jnp.dot(iy.astype(jnp.float32), Tm,
                          preferred_element_type=jnp.float32, precision=lax.Precision.HIGHEST))
        svals, slanes = [], []
        for _ in range(_K):
            m = jnp.max(all_s, axis=1, keepdims=True)
            am = jnp.min(jnp.where(all_s == m, iota64, _K * _K),
                         axis=1, keepdims=True)
            eid = jnp.min(jnp.where(iota64 == am, eidf, 3.0e4),
                          axis=1, keepdims=True)
            svals.append(m)
            slanes.append(eid)
            all_s = jnp.where(iota64 == am, -1e30, all_s)
        s_cols.append(jnp.concatenate(svals, axis=1))
        i_cols.append(jnp.concatenate(slanes, axis=1))
    s_out[...] = jnp.concatenate(s_cols, axis=1)
    i_out[...] = jnp.concatenate(i_cols, axis=1).astype(jnp.int32)


def _routing(x_flat, bn_gamma, bn_beta, W_q, kt):
    n_tokens = x_flat.shape[0]
    grid = n_tokens // _TBLK
    scores, indices = pl.pallas_call(
        _routing_body,
        grid=(grid,),
        in_specs=[
            pl.BlockSpec((_TBLK, _HIDDEN), lambda i: (i, 0)),
            pl.BlockSpec((_HIDDEN,), lambda i: (0,)),
            pl.BlockSpec((_HIDDEN,), lambda i: (0,)),
            pl.BlockSpec((_HIDDEN, 2 * _NUM_HEADS * _KEY_DIMS),
                         lambda i: (0, 0)),
            pl.BlockSpec((2 * _NUM_HEADS * _KEY_DIMS, _NUM_KEYS),
                         lambda i: (0, 0)),
        ],
        out_specs=[
            pl.BlockSpec((_TBLK, _KH), lambda i: (i, 0)),
            pl.BlockSpec((_TBLK, _KH), lambda i: (i, 0)),
        ],
        out_shape=[
            jax.ShapeDtypeStruct((n_tokens, _KH), jnp.float32),
            jax.ShapeDtypeStruct((n_tokens, _KH), jnp.int32),
        ],
    )(x_flat, bn_gamma, bn_beta, W_q, kt)
    return scores, indices


_NCHUNK = 4


def kernel(inputs, bn_gamma, bn_beta, W_q, keys, down, up):
    b, n, d = inputs.shape
    n_tokens = b * n
    x_flat = inputs.reshape(n_tokens, d)
    kt = jnp.transpose(keys, (2, 0, 3, 1)).reshape(2 * _NUM_HEADS * _KEY_DIMS,
                                                   _NUM_KEYS)
    down16 = lax.bitcast_convert_type(
        down.astype(jnp.bfloat16).reshape(_NUM_EXPERTS, _HIDDEN // 2, 2),
        jnp.int32)
    up16 = lax.bitcast_convert_type(
        up.astype(jnp.bfloat16).reshape(_NUM_EXPERTS, _HIDDEN // 2, 2),
        jnp.int32)
    x_shuf = x_flat.reshape(n_tokens, _HIDDEN // 32, 16, 2)
    x_shuf = jnp.transpose(x_shuf, (0, 1, 3, 2)).reshape(n_tokens, _HIDDEN)
    csz = n_tokens // _NCHUNK
    sc_comb = _make_sc_combine(csz)
    outs = []
    for c in range(_NCHUNK):
        xr = lax.slice(x_flat, (c * csz, 0), ((c + 1) * csz, d))
        xs = lax.slice(x_shuf, (c * csz, 0), ((c + 1) * csz, d))
        s_c, i_c = _routing(xr, bn_gamma, bn_beta, W_q, kt)
        outs.append(sc_comb(xs, i_c, s_c, down16, up16))
    out_s = jnp.concatenate(outs, axis=0)
    out = out_s.reshape(n_tokens, _HIDDEN // 32, 2, 16)
    out = jnp.transpose(out, (0, 1, 3, 2)).reshape(n_tokens, _HIDDEN)
    return out.reshape(b, n, d)


# speedup vs baseline: 1.6028x; 1.6028x over previous
import functools

import jax
import jax.numpy as jnp
from jax import lax
from jax.experimental import pallas as pl
from jax.experimental.pallas import tpu as pltpu
from jax.experimental.pallas import tpu_sc as plsc

_HIDDEN = 1024
_NUM_EXPERTS = 16384
_KEY_DIMS = 128
_K = 8
_NUM_HEADS = 8
_NUM_KEYS = 128
_KH = _NUM_HEADS * _K
_KC = 32
_NCH = _KH // _KC
_L = 16
_DC = _HIDDEN // _L


def _gelu16(x):
    z = jnp.abs(x) * 0.7071067811865476
    t = 1.0 / (1.0 + 0.3275911 * z)
    poly = ((((1.061405429 * t - 1.453152027) * t + 1.421413741) * t
             - 0.284496736) * t + 0.254829592) * t
    erf_abs = 1.0 - poly * jnp.exp(-z * z)
    erf = jnp.where(x >= 0.0, erf_abs, -erf_abs)
    return x * 0.5 * (1.0 + erf)


_NUM_SC = 2
_NUM_SUBCORES = 16


def _make_sc_combine(n_tokens):
    nw = _NUM_SC * _NUM_SUBCORES
    tpw = n_tokens // nw
    mesh = plsc.VectorSubcoreMesh(core_axis_name="c", subcore_axis_name="s",
                                  num_cores=_NUM_SC, num_subcores=_NUM_SUBCORES)

    @functools.partial(
        pl.kernel,
        out_type=jax.ShapeDtypeStruct((n_tokens, _HIDDEN), jnp.float32),
        mesh=mesh,
        compiler_params=pltpu.CompilerParams(needs_layout_passes=False),
        scratch_types=[
            pltpu.VMEM((2, _HIDDEN), jnp.float32),
            pltpu.VMEM((2, _KH), jnp.int32),
            pltpu.VMEM((2, _KH), jnp.float32),
            pltpu.VMEM((_KH,), jnp.float32),
            pltpu.VMEM((_KH,), jnp.float32),
            pltpu.VMEM((_KC, _HIDDEN), jnp.float32),
            pltpu.VMEM((_KC, _HIDDEN), jnp.float32),
            pltpu.VMEM((2, _HIDDEN), jnp.float32),
            pltpu.SemaphoreType.DMA,
            pltpu.SemaphoreType.DMA,
            pltpu.SemaphoreType.DMA,
            pltpu.SemaphoreType.DMA,
        ],
    )
    def body(x_hbm, idx_hbm, s_hbm, down_hbm, up_hbm, out_hbm,
             x2, i2, s2, t_v, act_v, rowA, rowB, acc2,
             semA, semB, semP, semO):
        wid = lax.axis_index("s") * _NUM_SC + lax.axis_index("c")
        base = wid * tpw
        lane0 = lax.iota(jnp.int32, _L) == 0

        def wait_row(sem, row):
            pltpu.make_async_copy(down_hbm.at[pl.ds(0, _KC)], row, sem).wait()

        def dots(row, xbuf, par, h):
            for g in range(_KC // 8):

                def dot_body(dj, accs, g=g):
                    sl = pl.ds(dj * _L, _L)
                    xc = xbuf[par, sl]
                    return tuple(
                        accs[r] + xc * row[g * 8 + r, sl]
                        for r in range(8))

                accs = lax.fori_loop(
                    0, _DC, dot_body,
                    tuple(jnp.zeros((_L,), jnp.float32) for _ in range(8)))
                for r in range(8):
                    k = h * _KC + g * 8 + r
                    t = jnp.sum(accs[r])
                    plsc.store_scatter(
                        t_v, [jnp.full((_L,), k, jnp.int32)],
                        jnp.full((_L,), t, jnp.float32), mask=lane0)

        def combine(row, par, h):
            act_a = act_v[pl.ds(h * _KC, _L)]
            act_b = act_v[pl.ds(h * _KC + _L, _L)]

            def comb_body(dc, _, h=h, act_a=act_a, act_b=act_b):
                sl = pl.ds(dc * _L, _L)
                if h == 0:
                    a = jnp.zeros((_L,), jnp.float32)
                else:
                    a = acc2[par, sl]
                for kk in range(_L):
                    a = a + act_a[kk] * row[kk, sl]
                for kk in range(_L):
                    a = a + act_b[kk] * row[_L + kk, sl]
                acc2[par, sl] = a
                return 0

            lax.fori_loop(0, _DC, comb_body, 0)

        pltpu.sync_copy(x_hbm.at[base], x2.at[0])
        pltpu.sync_copy(s_hbm.at[base], s2.at[0])
        pltpu.sync_copy(idx_hbm.at[base], i2.at[0])
        pltpu.async_copy(down_hbm.at[i2.at[0, pl.ds(0, _KC)]], rowA, semA)
        pltpu.async_copy(down_hbm.at[i2.at[0, pl.ds(_KC, _KC)]], rowB, semB)

        def token_body(i, _):
            par = lax.rem(i, 2)
            nxt = 1 - par
            tok = base + i
            tok1 = base + jnp.minimum(i + 1, tpw - 1)

            pltpu.async_copy(x_hbm.at[tok1], x2.at[nxt], semP)
            pltpu.async_copy(s_hbm.at[tok1], s2.at[nxt], semP)
            pltpu.async_copy(idx_hbm.at[tok1], i2.at[nxt], semP)

            wait_row(semA, rowA)
            dots(rowA, x2, par, 0)
            pltpu.async_copy(up_hbm.at[i2.at[par, pl.ds(0, _KC)]], rowA, semA)
            wait_row(semB, rowB)
            dots(rowB, x2, par, 1)
            pltpu.async_copy(up_hbm.at[i2.at[par, pl.ds(_KC, _KC)]], rowB,
                             semB)

            for c in range(_KH // _L):
                sl = pl.ds(c * _L, _L)
                s16 = s2[par, sl]
                sig = 1.0 / (1.0 + jnp.exp(-s16))
                act_v[sl] = sig * _gelu16(t_v[sl])

            pltpu.make_async_copy(x_hbm.at[tok1], x2.at[nxt], semP).wait()
            pltpu.make_async_copy(s_hbm.at[tok1], s2.at[nxt], semP).wait()
            pltpu.make_async_copy(idx_hbm.at[tok1], i2.at[nxt], semP).wait()

            @pl.when(i >= 2)
            def _():
                pltpu.make_async_copy(acc2.at[par], out_hbm.at[tok],
                                      semO).wait()

            wait_row(semA, rowA)
            combine(rowA, par, 0)
            pltpu.async_copy(down_hbm.at[i2.at[nxt, pl.ds(0, _KC)]], rowA,
                             semA)
            wait_row(semB, rowB)
            combine(rowB, par, 1)
            pltpu.async_copy(down_hbm.at[i2.at[nxt, pl.ds(_KC, _KC)]], rowB,
                             semB)

            pltpu.async_copy(acc2.at[par], out_hbm.at[tok], semO)
            return 0

        lax.fori_loop(0, tpw, token_body, 0)

        wait_row(semA, rowA)
        wait_row(semB, rowB)
        pltpu.make_async_copy(acc2.at[0], out_hbm.at[base], semO).wait()
        pltpu.make_async_copy(acc2.at[1], out_hbm.at[base], semO).wait()

    return body


_TBLK = 256


def _top8_iter(s, iota_l, nl):
    vals, lanes = [], []
    for _ in range(_K):
        m = jnp.max(s, axis=1, keepdims=True)
        am = jnp.min(jnp.where(s == m, iota_l, nl), axis=1, keepdims=True)
        vals.append(m)
        lanes.append(am)
        s = jnp.where(iota_l == am, -1e30, s)
    return jnp.concatenate(vals, axis=1), jnp.concatenate(lanes, axis=1)


def _routing_body(x_ref, g_ref, b_ref, wq_ref, kt_ref, s_out, i_out):
    rsqrt_1p = 0.9999950000374997
    x = x_ref[...] * (g_ref[...] * rsqrt_1p) + b_ref[...]
    h_mat = jnp.dot(x, wq_ref[...], preferred_element_type=jnp.float32,
                    precision=lax.Precision.DEFAULT)

    iota128 = lax.broadcasted_iota(jnp.int32, (_TBLK, _NUM_KEYS), 1)
    iota64 = lax.broadcasted_iota(jnp.int32, (_TBLK, _K * _K), 1)
    col = lax.broadcasted_iota(jnp.int32, (_K, _K * _K), 1)
    row = lax.broadcasted_iota(jnp.int32, (_K, _K * _K), 0)
    Rm = (col // _K == row).astype(jnp.float32)
    Tm = (col % _K == row).astype(jnp.float32)

    s_cols, i_cols = [], []
    for hd in range(_NUM_HEADS):
        per_plane = []
        for p in range(2):
            q_ph = h_mat[:, p * _NUM_HEADS * _KEY_DIMS + hd * _KEY_DIMS:
                         p * _NUM_HEADS * _KEY_DIMS + (hd + 1) * _KEY_DIMS]
            kslab = kt_ref[pl.ds((p * _NUM_HEADS + hd) * _KEY_DIMS,
                                 _KEY_DIMS), :]
            sim = jnp.dot(q_ph, kslab, preferred_element_type=jnp.float32,
                          precision=lax.Precision.DEFAULT)
            per_plane.append(_top8_iter(sim, iota128, _NUM_KEYS))
        (sx, ix), (sy, iy) = per_plane
        all_s = (jnp.dot(sx, Rm, preferred_element_type=jnp.float32, precision=lax.Precision.HIGHEST)
                 + jnp.dot(sy, Tm, preferred_element_type=jnp.float32, precision=lax.Precision.HIGHEST))
        eidf = (jnp.dot(ix.astype(jnp.float32), Rm,
                        preferred_element_type=jnp.float32, precision=lax.Precision.HIGHEST) * _NUM_KEYS
                + jnp.dot(iy.astype(jnp.float32), Tm,
                          preferred_element_type=jnp.float32, precision=lax.Precision.HIGHEST))
        svals, slanes = [], []
        for _ in range(_K):
            m = jnp.max(all_s, axis=1, keepdims=True)
            am = jnp.min(jnp.where(all_s == m, iota64, _K * _K),
                         axis=1, keepdims=True)
            eid = jnp.min(jnp.where(iota64 == am, eidf, 3.0e4),
                          axis=1, keepdims=True)
            svals.append(m)
            slanes.append(eid)
            all_s = jnp.where(iota64 == am, -1e30, all_s)
        s_cols.append(jnp.concatenate(svals, axis=1))
        i_cols.append(jnp.concatenate(slanes, axis=1))
    s_out[...] = jnp.concatenate(s_cols, axis=1)
    i_out[...] = jnp.concatenate(i_cols, axis=1).astype(jnp.int32)


def _routing(x_flat, bn_gamma, bn_beta, W_q, kt):
    n_tokens = x_flat.shape[0]
    grid = n_tokens // _TBLK
    scores, indices = pl.pallas_call(
        _routing_body,
        grid=(grid,),
        in_specs=[
            pl.BlockSpec((_TBLK, _HIDDEN), lambda i: (i, 0)),
            pl.BlockSpec((_HIDDEN,), lambda i: (0,)),
            pl.BlockSpec((_HIDDEN,), lambda i: (0,)),
            pl.BlockSpec((_HIDDEN, 2 * _NUM_HEADS * _KEY_DIMS),
                         lambda i: (0, 0)),
            pl.BlockSpec((2 * _NUM_HEADS * _KEY_DIMS, _NUM_KEYS),
                         lambda i: (0, 0)),
        ],
        out_specs=[
            pl.BlockSpec((_TBLK, _KH), lambda i: (i, 0)),
            pl.BlockSpec((_TBLK, _KH), lambda i: (i, 0)),
        ],
        out_shape=[
            jax.ShapeDtypeStruct((n_tokens, _KH), jnp.float32),
            jax.ShapeDtypeStruct((n_tokens, _KH), jnp.int32),
        ],
    )(x_flat, bn_gamma, bn_beta, W_q, kt)
    return scores, indices


_NCHUNK = 4


def kernel(inputs, bn_gamma, bn_beta, W_q, keys, down, up):
    b, n, d = inputs.shape
    n_tokens = b * n
    x_flat = inputs.reshape(n_tokens, d)
    kt = jnp.transpose(keys, (2, 0, 3, 1)).reshape(2 * _NUM_HEADS * _KEY_DIMS,
                                                   _NUM_KEYS)
    csz = n_tokens // _NCHUNK
    sc_comb = _make_sc_combine(csz)
    outs = []
    for c in range(_NCHUNK):
        xs = lax.slice(x_flat, (c * csz, 0), ((c + 1) * csz, d))
        s_c, i_c = _routing(xs, bn_gamma, bn_beta, W_q, kt)
        outs.append(sc_comb(xs, i_c, s_c, down, up))
    return jnp.concatenate(outs, axis=0).reshape(b, n, d)
